# baseline (speedup 1.0000x reference)
import functools

import jax
import jax.numpy as jnp
from jax import lax
from jax.experimental import pallas as pl
from jax.experimental.pallas import tpu as pltpu
from jax.experimental.pallas import tpu_sc as plsc

B, F, D = 4096, 26, 16
SEG = 100000
SEGP = SEG + 96
TOTAL = F * SEG
NW = 32
FD = F * D
TPW = FD // NW
H1, H2 = 256, 128
BLK = 4096


def _sc_gather(xt, emb_t, lin_t):
    mesh = plsc.VectorSubcoreMesh(core_axis_name="c", subcore_axis_name="s")

    @functools.partial(
        pl.kernel,
        mesh=mesh,
        out_type=[
            jax.ShapeDtypeStruct((FD, B), jnp.float32),
            jax.ShapeDtypeStruct((F, B), jnp.float32),
        ],
        scratch_types=[
            pltpu.VMEM((B,), jnp.int32),
            pltpu.VMEM((SEGP,), jnp.float32),
            pltpu.VMEM((B,), jnp.float32),
        ],
        compiler_params=pltpu.CompilerParams(needs_layout_passes=False),
    )
    def k(xt_hbm, emb_hbm, lin_hbm, emb_out, lin_out, ids_v, seg_v, out_v):
        wid = lax.axis_index("s") * 2 + lax.axis_index("c")

        def pick_all(shift):
            def body(i, _):
                idx = ids_v[pl.ds(i * 16, 16)] + shift
                out_v[pl.ds(i * 16, 16)] = plsc.load_gather(seg_v, [idx])
                return 0
            lax.fori_loop(0, B // 16, body, 0)

        def seg_start(f):
            a = f * SEG
            sa = pl.multiple_of(a - lax.rem(a, 128), 128)
            return sa, a - sa

        for j in range(TPW):
            t = wid * TPW + j
            f = t // D
            d = t % D
            sa, shift = seg_start(f)
            pltpu.sync_copy(xt_hbm.at[f], ids_v)
            pltpu.sync_copy(emb_hbm.at[d, pl.ds(sa, SEGP)], seg_v)
            pick_all(shift)
            pltpu.sync_copy(out_v, emb_out.at[t])

        @pl.when(wid < F)
        def _():
            sa, shift = seg_start(wid)
            pltpu.sync_copy(xt_hbm.at[wid], ids_v)
            pltpu.sync_copy(lin_hbm.at[0, pl.ds(sa, SEGP)], seg_v)
            pick_all(shift)
            pltpu.sync_copy(out_v, lin_out.at[wid])

    return k(xt, emb_t, lin_t)


def _tc_body(emb_ref, lin_ref, W1_ref, s1_ref, t1_ref, W2_ref, s2_ref,
             t2_ref, w3_ref, cb_ref, out_ref):
    et = emb_ref[...]
    r = lax.broadcasted_iota(jnp.int32, (D, FD), 1)
    c = lax.broadcasted_iota(jnp.int32, (D, FD), 0)
    sel = jnp.where((r % D) == c, 1.0, 0.0)
    sum_e = jnp.dot(sel, et, preferred_element_type=jnp.float32)
    t1 = jnp.sum(sum_e * sum_e, axis=0, keepdims=True)
    t2 = jnp.sum(et * et, axis=0, keepdims=True)
    second = 0.5 * (t1 - t2)
    first = jnp.sum(lin_ref[...], axis=0, keepdims=True)
    dn = (((0,), (0,)), ((), ()))
    h = lax.dot_general(W1_ref[...], et, dn,
                        preferred_element_type=jnp.float32)
    h = jnp.maximum(h * s1_ref[...] + t1_ref[...], 0.0)
    h = lax.dot_general(W2_ref[...], h, dn,
                        preferred_element_type=jnp.float32)
    h = jnp.maximum(h * s2_ref[...] + t2_ref[...], 0.0)
    deep = lax.dot_general(w3_ref[...], h, dn,
                           preferred_element_type=jnp.float32)
    out_ref[...] = first + second + deep + cb_ref[0, 0]


def _tc_dense(emb_t, lin_t, W1, s1, t1, W2, s2, t2, w3, cb):
    grid = (B // BLK,)
    full = lambda shape: pl.BlockSpec(shape, lambda i: (0, 0))
    return pl.pallas_call(
        _tc_body,
        grid=grid,
        in_specs=[
            pl.BlockSpec((FD, BLK), lambda i: (0, i)),
            pl.BlockSpec((F, BLK), lambda i: (0, i)),
            full((FD, H1)),
            full((H1, 1)),
            full((H1, 1)),
            full((H1, H2)),
            full((H2, 1)),
            full((H2, 1)),
            full((H2, 1)),
            full((1, 1)),
        ],
        out_specs=pl.BlockSpec((1, BLK), lambda i: (0, i)),
        out_shape=jax.ShapeDtypeStruct((1, B), jnp.float32),
    )(emb_t, lin_t, W1, s1, t1, W2, s2, t2, w3, cb)


def kernel(x, lin_w, lin_b, emb_w, W1, b1, g1, be1, W2, b2, g2, be2, W3, b3):
    xt = x.T
    emb_t = emb_w.T
    lin_t = lin_w.T

    emb_feat, lin_feat = _sc_gather(xt, emb_t, lin_t)

    inv = 1.0 / jnp.sqrt(jnp.float32(1.0 + 1e-5))
    s1 = (g1 * inv).reshape(H1, 1)
    t1 = (b1 * g1 * inv + be1).reshape(H1, 1)
    s2 = (g2 * inv).reshape(H2, 1)
    t2 = (b2 * g2 * inv + be2).reshape(H2, 1)
    w3 = W3
    cb = (lin_b + b3).reshape(1, 1)

    out = _tc_dense(emb_feat, lin_feat, W1, s1, t1, W2, s2, t2, w3, cb)
    return out.reshape(B)


# speedup vs baseline: 1.0846x; 1.0846x over previous
import functools

import jax
import jax.numpy as jnp
from jax import lax
from jax.experimental import pallas as pl
from jax.experimental.pallas import tpu as pltpu
from jax.experimental.pallas import tpu_sc as plsc

B, F, D = 4096, 26, 16
SEG = 100000
SEGP = SEG + 96
TOTAL = F * SEG
NW = 32
FD = F * D
TPW = FD // NW
H1, H2 = 256, 128
BLK = 2048


def _sc_gather(xt, emb_t, lin_t):
    mesh = plsc.VectorSubcoreMesh(core_axis_name="c", subcore_axis_name="s")

    @functools.partial(
        pl.kernel,
        mesh=mesh,
        out_type=[
            jax.ShapeDtypeStruct((FD, B), jnp.float32),
            jax.ShapeDtypeStruct((F, B), jnp.float32),
        ],
        scratch_types=[
            pltpu.VMEM((B,), jnp.int32),
            pltpu.VMEM((SEGP,), jnp.float32),
            pltpu.VMEM((B,), jnp.float32),
        ],
        compiler_params=pltpu.CompilerParams(needs_layout_passes=False),
    )
    def k(xt_hbm, emb_hbm, lin_hbm, emb_out, lin_out, ids_v, seg_v, out_v):
        wid = lax.axis_index("s") * 2 + lax.axis_index("c")

        def pick_all(shift):
            def body(i, _):
                idx = ids_v[pl.ds(i * 16, 16)] + shift
                out_v[pl.ds(i * 16, 16)] = plsc.load_gather(seg_v, [idx])
                return 0
            lax.fori_loop(0, B // 16, body, 0)

        def seg_start(f):
            a = f * SEG
            sa = pl.multiple_of(a - lax.rem(a, 128), 128)
            return sa, a - sa

        for j in range(TPW):
            t = wid * TPW + j
            f = t // D
            d = t % D
            sa, shift = seg_start(f)
            if j == 0:
                pltpu.sync_copy(xt_hbm.at[f], ids_v)
            else:
                @pl.when(d == 0)
                def _():
                    pltpu.sync_copy(xt_hbm.at[f], ids_v)
            pltpu.sync_copy(emb_hbm.at[d, pl.ds(sa, SEGP)], seg_v)
            pick_all(shift)
            pltpu.sync_copy(out_v, emb_out.at[t])

        @pl.when(wid < F)
        def _():
            sa, shift = seg_start(wid)
            pltpu.sync_copy(xt_hbm.at[wid], ids_v)
            pltpu.sync_copy(lin_hbm.at[0, pl.ds(sa, SEGP)], seg_v)
            pick_all(shift)
            pltpu.sync_copy(out_v, lin_out.at[wid])

    return k(xt, emb_t, lin_t)


def _tc_body(emb_ref, lin_ref, W1_ref, s1_ref, t1_ref, W2_ref, s2_ref,
             t2_ref, w3_ref, cb_ref, out_ref):
    et = emb_ref[...]
    r = lax.broadcasted_iota(jnp.int32, (D, FD), 1)
    c = lax.broadcasted_iota(jnp.int32, (D, FD), 0)
    sel = jnp.where((r % D) == c, 1.0, 0.0)
    sum_e = jnp.dot(sel, et, preferred_element_type=jnp.float32)
    t1 = jnp.sum(sum_e * sum_e, axis=0, keepdims=True)
    t2 = jnp.sum(et * et, axis=0, keepdims=True)
    second = 0.5 * (t1 - t2)
    first = jnp.sum(lin_ref[...], axis=0, keepdims=True)
    dn = (((0,), (0,)), ((), ()))
    h = lax.dot_general(W1_ref[...], et, dn,
                        preferred_element_type=jnp.float32)
    h = jnp.maximum(h * s1_ref[...] + t1_ref[...], 0.0)
    h = lax.dot_general(W2_ref[...], h, dn,
                        preferred_element_type=jnp.float32)
    h = jnp.maximum(h * s2_ref[...] + t2_ref[...], 0.0)
    deep = lax.dot_general(w3_ref[...], h, dn,
                           preferred_element_type=jnp.float32)
    out_ref[...] = first + second + deep + cb_ref[0, 0]


def _tc_dense(emb_t, lin_t, W1, s1, t1, W2, s2, t2, w3, cb):
    grid = (B // BLK,)
    full = lambda shape: pl.BlockSpec(shape, lambda i: (0, 0))
    return pl.pallas_call(
        _tc_body,
        grid=grid,
        in_specs=[
            pl.BlockSpec((FD, BLK), lambda i: (0, i)),
            pl.BlockSpec((F, BLK), lambda i: (0, i)),
            full((FD, H1)),
            full((H1, 1)),
            full((H1, 1)),
            full((H1, H2)),
            full((H2, 1)),
            full((H2, 1)),
            full((H2, 1)),
            full((1, 1)),
        ],
        out_specs=pl.BlockSpec((1, BLK), lambda i: (0, i)),
        out_shape=jax.ShapeDtypeStruct((1, B), jnp.float32),
    )(emb_t, lin_t, W1, s1, t1, W2, s2, t2, w3, cb)


def kernel(x, lin_w, lin_b, emb_w, W1, b1, g1, be1, W2, b2, g2, be2, W3, b3):
    xt = x.T
    emb_t = emb_w.T
    lin_t = lin_w.T

    emb_feat, lin_feat = _sc_gather(xt, emb_t, lin_t)

    inv = 1.0 / jnp.sqrt(jnp.float32(1.0 + 1e-5))
    s1 = (g1 * inv).reshape(H1, 1)
    t1 = (b1 * g1 * inv + be1).reshape(H1, 1)
    s2 = (g2 * inv).reshape(H2, 1)
    t2 = (b2 * g2 * inv + be2).reshape(H2, 1)
    w3 = W3
    cb = (lin_b + b3).reshape(1, 1)

    out = _tc_dense(emb_feat, lin_feat, W1, s1, t1, W2, s2, t2, w3, cb)
    return out.reshape(B)


# speedup vs baseline: 1.1620x; 1.0714x over previous
import functools

import jax
import jax.numpy as jnp
from jax import lax
from jax.experimental import pallas as pl
from jax.experimental.pallas import tpu as pltpu
from jax.experimental.pallas import tpu_sc as plsc

B, F, D = 4096, 26, 16
SEG = 100000
SEGP = SEG + 96
HSEG = SEGP // 2
TOTAL = F * SEG
NW = 32
FD = F * D
TPW = FD // NW
H1, H2 = 256, 128
BLK = 2048


def _sc_gather(xt, emb_t, lin_t):
    mesh = plsc.VectorSubcoreMesh(core_axis_name="c", subcore_axis_name="s")

    @functools.partial(
        pl.kernel,
        mesh=mesh,
        out_type=[
            jax.ShapeDtypeStruct((FD, B), jnp.float32),
            jax.ShapeDtypeStruct((F, B), jnp.float32),
        ],
        scratch_types=[
            pltpu.VMEM((B,), jnp.int32),
            [pltpu.VMEM((HSEG,), jnp.float32) for _ in range(2)],
            [pltpu.VMEM((B,), jnp.float32) for _ in range(2)],
            [pltpu.SemaphoreType.DMA for _ in range(4)],
        ],
        compiler_params=pltpu.CompilerParams(needs_layout_passes=False),
    )
    def k(xt_hbm, emb_hbm, lin_hbm, emb_out, lin_out, ids_v, segs, outs,
          sems):
        wid = lax.axis_index("s") * 2 + lax.axis_index("c")

        def pick_half0(shift, seg, out):
            def body(i, _):
                idx = ids_v[pl.ds(i * 16, 16)] + shift
                i0 = jnp.minimum(idx, HSEG - 1)
                out[pl.ds(i * 16, 16)] = plsc.load_gather(seg, [i0])
                return 0
            lax.fori_loop(0, B // 16, body, 0)

        def pick_half1(shift, seg, out):
            def body(i, _):
                sl = pl.ds(i * 16, 16)
                idx = ids_v[sl] + shift
                i1 = jnp.maximum(idx - HSEG, 0)
                g1 = plsc.load_gather(seg, [i1])
                out[sl] = jnp.where(idx >= HSEG, g1, out[sl])
                return 0
            lax.fori_loop(0, B // 16, body, 0)

        def task_params(j):
            t = wid * TPW + j
            f = t // D
            d = lax.rem(t, D)
            a = f * SEG
            sa = pl.multiple_of(a - lax.rem(a, 128), 128)
            return t, f, d, sa, a - sa

        def issue(j, h):
            _, _, d, sa, _ = task_params(j)
            return pltpu.async_copy(
                emb_hbm.at[d, pl.ds(sa + h * HSEG, HSEG)], segs[h], sems[h])

        seg_cp = [issue(0, 0), issue(0, 1)]
        out_cp = [None, None]
        for j in range(TPW):
            t, f, d, _, shift = task_params(j)
            if j == 0:
                pltpu.sync_copy(xt_hbm.at[f], ids_v)
            else:
                @pl.when(d == 0)
                def _():
                    pltpu.sync_copy(xt_hbm.at[f], ids_v)
            if out_cp[j % 2] is not None:
                out_cp[j % 2].wait()
            out_v = outs[j % 2]
            seg_cp[0].wait()
            pick_half0(shift, segs[0], out_v)
            if j + 1 < TPW:
                seg_cp[0] = issue(j + 1, 0)
            seg_cp[1].wait()
            pick_half1(shift, segs[1], out_v)
            if j + 1 < TPW:
                seg_cp[1] = issue(j + 1, 1)
            out_cp[j % 2] = pltpu.async_copy(out_v, emb_out.at[t],
                                             sems[2 + j % 2])
        out_cp[0].wait()
        out_cp[1].wait()

        @pl.when(wid < F)
        def _():
            a = wid * SEG
            sa = pl.multiple_of(a - lax.rem(a, 128), 128)
            shift = a - sa
            pltpu.sync_copy(xt_hbm.at[wid], ids_v)
            pltpu.sync_copy(lin_hbm.at[0, pl.ds(sa, HSEG)], segs[0])
            pltpu.sync_copy(lin_hbm.at[0, pl.ds(sa + HSEG, HSEG)], segs[1])
            pick_half0(shift, segs[0], outs[0])
            pick_half1(shift, segs[1], outs[0])
            pltpu.sync_copy(outs[0], lin_out.at[wid])

    return k(xt, emb_t, lin_t)


def _tc_body(emb_ref, lin_ref, W1_ref, s1_ref, t1_ref, W2_ref, s2_ref,
             t2_ref, w3_ref, cb_ref, out_ref):
    et = emb_ref[...]
    r = lax.broadcasted_iota(jnp.int32, (D, FD), 1)
    c = lax.broadcasted_iota(jnp.int32, (D, FD), 0)
    sel = jnp.where((r % D) == c, 1.0, 0.0)
    sum_e = jnp.dot(sel, et, preferred_element_type=jnp.float32)
    t1 = jnp.sum(sum_e * sum_e, axis=0, keepdims=True)
    t2 = jnp.sum(et * et, axis=0, keepdims=True)
    second = 0.5 * (t1 - t2)
    first = jnp.sum(lin_ref[...], axis=0, keepdims=True)
    dn = (((0,), (0,)), ((), ()))
    h = lax.dot_general(W1_ref[...], et, dn,
                        preferred_element_type=jnp.float32)
    h = jnp.maximum(h * s1_ref[...] + t1_ref[...], 0.0)
    h = lax.dot_general(W2_ref[...], h, dn,
                        preferred_element_type=jnp.float32)
    h = jnp.maximum(h * s2_ref[...] + t2_ref[...], 0.0)
    deep = lax.dot_general(w3_ref[...], h, dn,
                           preferred_element_type=jnp.float32)
    out_ref[...] = first + second + deep + cb_ref[0, 0]


def _tc_dense(emb_t, lin_t, W1, s1, t1, W2, s2, t2, w3, cb):
    grid = (B // BLK,)
    full = lambda shape: pl.BlockSpec(shape, lambda i: (0, 0))
    return pl.pallas_call(
        _tc_body,
        grid=grid,
        in_specs=[
            pl.BlockSpec((FD, BLK), lambda i: (0, i)),
            pl.BlockSpec((F, BLK), lambda i: (0, i)),
            full((FD, H1)),
            full((H1, 1)),
            full((H1, 1)),
            full((H1, H2)),
            full((H2, 1)),
            full((H2, 1)),
            full((H2, 1)),
            full((1, 1)),
        ],
        out_specs=pl.BlockSpec((1, BLK), lambda i: (0, i)),
        out_shape=jax.ShapeDtypeStruct((1, B), jnp.float32),
    )(emb_t, lin_t, W1, s1, t1, W2, s2, t2, w3, cb)


def kernel(x, lin_w, lin_b, emb_w, W1, b1, g1, be1, W2, b2, g2, be2, W3, b3):
    xt = x.T
    emb_t = emb_w.T
    lin_t = lin_w.T

    emb_feat, lin_feat = _sc_gather(xt, emb_t, lin_t)

    inv = 1.0 / jnp.sqrt(jnp.float32(1.0 + 1e-5))
    s1 = (g1 * inv).reshape(H1, 1)
    t1 = (b1 * g1 * inv + be1).reshape(H1, 1)
    s2 = (g2 * inv).reshape(H2, 1)
    t2 = (b2 * g2 * inv + be2).reshape(H2, 1)
    w3 = W3
    cb = (lin_b + b3).reshape(1, 1)

    out = _tc_dense(emb_feat, lin_feat, W1, s1, t1, W2, s2, t2, w3, cb)
    return out.reshape(B)
